# baseline (speedup 1.0000x reference)
import functools

import jax
import jax.numpy as jnp
from jax import lax
from jax.experimental import pallas as pl
from jax.experimental.pallas import tpu as pltpu
from jax.experimental.pallas import tpu_sc as plsc

_T = 2048
_D = 768
_E = 64
_H = 128
_S = 2048

_BLK = 128
_G = 80
_R = _G * _BLK

_NW = 32
_BPW = _T // _NW


_NS = 2
_SB = _S // _NS


def _plan_shared_body(x_ref, rw_ref, gw_ref, uw_ref, dw_ref,
                      dest_ref, be_ref, wgt_ref, sh_ref):
    s = pl.program_id(0)

    @pl.when(s == 0)
    def _plan_step():
        _plan_calc(x_ref, rw_ref, dest_ref, be_ref, wgt_ref)

    @pl.when(s > 0)
    def _shared_step():
        @pl.when(s == 1)
        def _init():
            sh_ref[...] = jnp.zeros_like(sh_ref)

        g = jnp.dot(x_ref[...], gw_ref[...],
                    preferred_element_type=jnp.float32)
        u = jnp.dot(x_ref[...], uw_ref[...],
                    preferred_element_type=jnp.float32)
        a = jax.nn.silu(g) * u
        sh_ref[...] += jnp.dot(a, dw_ref[...],
                               preferred_element_type=jnp.float32)


def _plan_calc(x_ref, rw_ref, dest_ref, be_ref, wgt_ref):
    logits = jnp.dot(x_ref[...], rw_ref[...],
                     preferred_element_type=jnp.float32)
    m = jnp.max(logits, axis=1, keepdims=True)
    lane = lax.broadcasted_iota(jnp.int32, (_T, _E), 1)
    eq = logits == m
    amin = jnp.min(jnp.where(eq, lane, _E), axis=1, keepdims=True)
    P = (lane == amin).astype(jnp.float32)
    wgt_ref[...] = jnp.broadcast_to(jax.nn.sigmoid(m), (_T, 16))

    row = lax.broadcasted_iota(jnp.int32, (_T, _T), 0)
    col = lax.broadcasted_iota(jnp.int32, (_T, _T), 1)
    ltri = (col <= row).astype(jnp.float32)
    csum = jnp.dot(ltri, P, preferred_element_type=jnp.float32)
    rank = jnp.sum(csum * P, axis=1, keepdims=True) - 1.0

    counts = csum[_T - 1:_T, :]
    nblk = jnp.floor((counts + (_BLK - 1.0)) / _BLK)
    ei = lax.broadcasted_iota(jnp.int32, (_E, _E), 0)
    ej = lax.broadcasted_iota(jnp.int32, (_E, _E), 1)
    stri = (ei < ej).astype(jnp.float32)
    bstart = jnp.dot(nblk, stri, preferred_element_type=jnp.float32)
    pad_start = _BLK * bstart

    dest = jnp.sum(P * pad_start, axis=1, keepdims=True) + rank
    dest_ref[...] = dest.astype(jnp.int32)

    gi = lax.broadcasted_iota(jnp.int32, (_G, _E), 0).astype(jnp.float32)
    bsb = jnp.broadcast_to(bstart, (_G, _E))
    be = (jnp.sum((bsb <= gi).astype(jnp.int32), axis=1, keepdims=True) - 1)
    total = jnp.sum(nblk, axis=1, keepdims=True).astype(jnp.int32)
    be_ref[...] = jnp.concatenate([be, total], axis=0)


def _plan_shared(xf, router_w, gw, uw, dw):
    return pl.pallas_call(
        _plan_shared_body,
        grid=(1 + _NS,),
        in_specs=[
            pl.BlockSpec((_T, _D), lambda s: (0, 0)),
            pl.BlockSpec((_D, _E), lambda s: (0, 0)),
            pl.BlockSpec((_D, _SB), lambda s: (0, jnp.maximum(s - 1, 0))),
            pl.BlockSpec((_D, _SB), lambda s: (0, jnp.maximum(s - 1, 0))),
            pl.BlockSpec((_SB, _D), lambda s: (jnp.maximum(s - 1, 0), 0)),
        ],
        out_specs=(
            pl.BlockSpec((_T, 1), lambda s: (0, 0)),
            pl.BlockSpec((_G + 1, 1), lambda s: (0, 0)),
            pl.BlockSpec((_T, 16), lambda s: (0, 0)),
            pl.BlockSpec((_T, _D), lambda s: (0, 0)),
        ),
        out_shape=(
            jax.ShapeDtypeStruct((_T, 1), jnp.int32),
            jax.ShapeDtypeStruct((_G + 1, 1), jnp.int32),
            jax.ShapeDtypeStruct((_T, 16), jnp.float32),
            jax.ShapeDtypeStruct((_T, _D), jnp.float32),
        ),
    )(xf, router_w, gw, uw, dw)


@functools.cache
def _sc_kernels():
    mesh = plsc.VectorSubcoreMesh(core_axis_name="c", subcore_axis_name="s")
    scratch = [
        pltpu.VMEM((_BPW,), jnp.int32),
        pltpu.VMEM((_BPW, _D), jnp.float32),
        pltpu.SemaphoreType.DMA,
    ]

    @functools.partial(
        pl.kernel,
        mesh=mesh,
        out_type=jax.ShapeDtypeStruct((_R, _D), jnp.float32),
        scratch_types=scratch,
    )
    def sc_scatter(x_hbm, dest_hbm, xpad_hbm, idx_v, rows_v, sem):
        wid = lax.axis_index("s") * 2 + lax.axis_index("c")
        base = wid * _BPW
        pltpu.sync_copy(dest_hbm.at[pl.ds(base, _BPW)], idx_v)
        pltpu.sync_copy(x_hbm.at[pl.ds(base, _BPW)], rows_v)
        pltpu.async_copy(rows_v, xpad_hbm.at[idx_v], sem).wait()

    @functools.partial(
        pl.kernel,
        mesh=mesh,
        out_type=jax.ShapeDtypeStruct((_T, _D), jnp.float32),
        scratch_types=[
            pltpu.VMEM((_BPW,), jnp.int32),
            pltpu.VMEM((_BPW, 16), jnp.float32),
            pltpu.VMEM((_BPW, _D), jnp.float32),
            pltpu.VMEM((_BPW, _D), jnp.float32),
            pltpu.SemaphoreType.DMA,
        ],
    )
    def sc_gather_combine(opad_hbm, dest_hbm, wgt_hbm, sh_hbm, out_hbm,
                          idx_v, wgt_v, moe_v, acc_v, sem):
        wid = lax.axis_index("s") * 2 + lax.axis_index("c")
        base = wid * _BPW
        pltpu.sync_copy(dest_hbm.at[pl.ds(base, _BPW)], idx_v)
        pltpu.sync_copy(wgt_hbm.at[pl.ds(base, _BPW)], wgt_v)
        pltpu.sync_copy(sh_hbm.at[pl.ds(base, _BPW)], acc_v)
        pltpu.async_copy(opad_hbm.at[idx_v], moe_v, sem).wait()

        def row_combine(t, carry):
            wv = wgt_v[t, :]
            for c in range(_D // 16):
                sl = (t, pl.ds(c * 16, 16))
                acc_v[sl] = acc_v[sl] + wv * moe_v[sl]
            return carry

        lax.fori_loop(0, _BPW, row_combine, 0)
        pltpu.sync_copy(acc_v, out_hbm.at[pl.ds(base, _BPW)])

    return sc_scatter, sc_gather_combine


def _sc_scatter(xf, dest_flat):
    return _sc_kernels()[0](xf, dest_flat)


def _sc_gather_combine(out_pad, dest_flat, wgt_flat, shared_out):
    return _sc_kernels()[1](out_pad, dest_flat, wgt_flat, shared_out)


def _gmm_body(be_ref, x_ref, gup_ref, dw_ref, out_ref):
    g = pl.program_id(0)

    @pl.when(g < be_ref[_G])
    def _compute():
        gu = jnp.dot(x_ref[...], gup_ref[0],
                     preferred_element_type=jnp.float32)
        act = jax.nn.silu(gu[:, :_H]) * gu[:, _H:]
        out_ref[...] = jnp.dot(act, dw_ref[0],
                               preferred_element_type=jnp.float32)


def _gmm(be, x_pad, gate_up_w, down_w):
    grid_spec = pltpu.PrefetchScalarGridSpec(
        num_scalar_prefetch=1,
        grid=(_G,),
        in_specs=[
            pl.BlockSpec((_BLK, _D),
                         lambda g, be_r: (jnp.minimum(g, be_r[_G] - 1), 0)),
            pl.BlockSpec((1, _D, 2 * _H),
                         lambda g, be_r: (be_r[jnp.minimum(g, be_r[_G] - 1)],
                                          0, 0)),
            pl.BlockSpec((1, _H, _D),
                         lambda g, be_r: (be_r[jnp.minimum(g, be_r[_G] - 1)],
                                          0, 0)),
        ],
        out_specs=pl.BlockSpec(
            (_BLK, _D), lambda g, be_r: (jnp.minimum(g, be_r[_G] - 1), 0)),
    )
    return pl.pallas_call(
        _gmm_body,
        grid_spec=grid_spec,
        out_shape=jax.ShapeDtypeStruct((_R, _D), jnp.float32),
    )(be, x_pad, gate_up_w, down_w)


@jax.jit
def kernel(x, router_w, gate_up_w, down_w, shared_gate_w, shared_up_w,
           shared_down_w):
    Bc, Tc, C = x.shape
    xf = x.reshape(Tc, C)

    dest, be, wgt, shared_out = _plan_shared(
        xf, router_w, shared_gate_w, shared_up_w, shared_down_w)
    dest_flat = dest.reshape(_T)
    be_flat = be.reshape(_G + 1)

    x_pad = _sc_scatter(xf, dest_flat)
    out_pad = _gmm(be_flat, x_pad, gate_up_w, down_w)
    out = _sc_gather_combine(out_pad, dest_flat, wgt, shared_out)
    return out.reshape(Bc, Tc, C)


# speedup vs baseline: 1.0591x; 1.0591x over previous
import functools

import jax
import jax.numpy as jnp
from jax import lax
from jax.experimental import pallas as pl
from jax.experimental.pallas import tpu as pltpu
from jax.experimental.pallas import tpu_sc as plsc

_T = 2048
_D = 768
_E = 64
_H = 128
_S = 2048

_BLK = 64
_G = 96
_R = _G * _BLK

_NW = 32
_BPW = _T // _NW


def _plan_body(x_ref, rw_ref, dest_ref, be_ref, wgt_ref):
    logits = jnp.dot(x_ref[...], rw_ref[...],
                     preferred_element_type=jnp.float32)
    m = jnp.max(logits, axis=1, keepdims=True)
    lane = lax.broadcasted_iota(jnp.int32, (_T, _E), 1)
    eq = logits == m
    amin = jnp.min(jnp.where(eq, lane, _E), axis=1, keepdims=True)
    P = (lane == amin).astype(jnp.float32)
    wgt_ref[...] = jax.nn.sigmoid(m)

    row = lax.broadcasted_iota(jnp.int32, (_T, _T), 0)
    col = lax.broadcasted_iota(jnp.int32, (_T, _T), 1)
    ltri = (col <= row).astype(jnp.float32)
    csum = jnp.dot(ltri, P, preferred_element_type=jnp.float32)
    rank = jnp.sum(csum * P, axis=1, keepdims=True) - 1.0

    counts = csum[_T - 1:_T, :]
    nblk = jnp.floor((counts + (_BLK - 1.0)) / _BLK)
    ei = lax.broadcasted_iota(jnp.int32, (_E, _E), 0)
    ej = lax.broadcasted_iota(jnp.int32, (_E, _E), 1)
    stri = (ei < ej).astype(jnp.float32)
    bstart = jnp.dot(nblk, stri, preferred_element_type=jnp.float32)
    pad_start = _BLK * bstart

    dest = jnp.sum(P * pad_start, axis=1, keepdims=True) + rank
    dest_ref[...] = dest.astype(jnp.int32)

    gi = lax.broadcasted_iota(jnp.int32, (_G, _E), 0).astype(jnp.float32)
    bsb = jnp.broadcast_to(bstart, (_G, _E))
    be = (jnp.sum((bsb <= gi).astype(jnp.int32), axis=1, keepdims=True) - 1)
    total = jnp.sum(nblk, axis=1, keepdims=True).astype(jnp.int32)
    be_ref[...] = jnp.concatenate([be, total], axis=0)


def _plan(xf, router_w):
    return pl.pallas_call(
        _plan_body,
        out_shape=(
            jax.ShapeDtypeStruct((_T, 1), jnp.int32),
            jax.ShapeDtypeStruct((_G + 1, 1), jnp.int32),
            jax.ShapeDtypeStruct((_T, 1), jnp.float32),
        ),
    )(xf, router_w)


@functools.cache
def _sc_kernels():
    mesh = plsc.VectorSubcoreMesh(core_axis_name="c", subcore_axis_name="s")
    scratch = [
        pltpu.VMEM((_BPW,), jnp.int32),
        pltpu.VMEM((_BPW, _D), jnp.float32),
        pltpu.SemaphoreType.DMA,
    ]

    @functools.partial(
        pl.kernel,
        mesh=mesh,
        out_type=jax.ShapeDtypeStruct((_R, _D), jnp.float32),
        scratch_types=scratch,
    )
    def sc_scatter(x_hbm, dest_hbm, xpad_hbm, idx_v, rows_v, sem):
        wid = lax.axis_index("s") * 2 + lax.axis_index("c")
        base = wid * _BPW
        pltpu.sync_copy(dest_hbm.at[pl.ds(base, _BPW)], idx_v)
        pltpu.sync_copy(x_hbm.at[pl.ds(base, _BPW)], rows_v)
        pltpu.async_copy(rows_v, xpad_hbm.at[idx_v], sem).wait()

    @functools.partial(
        pl.kernel,
        mesh=mesh,
        out_type=jax.ShapeDtypeStruct((_T, _D), jnp.float32),
        scratch_types=scratch,
    )
    def sc_gather(opad_hbm, dest_hbm, rows_hbm, idx_v, rows_v, sem):
        wid = lax.axis_index("s") * 2 + lax.axis_index("c")
        base = wid * _BPW
        pltpu.sync_copy(dest_hbm.at[pl.ds(base, _BPW)], idx_v)
        pltpu.async_copy(opad_hbm.at[idx_v], rows_v, sem).wait()
        pltpu.sync_copy(rows_v, rows_hbm.at[pl.ds(base, _BPW)])

    return sc_scatter, sc_gather


def _sc_scatter(xf, dest_flat):
    return _sc_kernels()[0](xf, dest_flat)


def _sc_gather(out_pad, dest_flat):
    return _sc_kernels()[1](out_pad, dest_flat)


def _gmm_body(be_ref, x_ref, gup_ref, dw_ref, out_ref):
    g = pl.program_id(0)

    @pl.when(g < be_ref[_G])
    def _compute():
        gu = jnp.dot(x_ref[...], gup_ref[0],
                     preferred_element_type=jnp.float32)
        act = jax.nn.silu(gu[:, :_H]) * gu[:, _H:]
        out_ref[...] = jnp.dot(act, dw_ref[0],
                               preferred_element_type=jnp.float32)


def _gmm(be, x_pad, gate_up_w, down_w):
    grid_spec = pltpu.PrefetchScalarGridSpec(
        num_scalar_prefetch=1,
        grid=(_G,),
        in_specs=[
            pl.BlockSpec((_BLK, _D),
                         lambda g, be_r: (jnp.minimum(g, be_r[_G] - 1), 0)),
            pl.BlockSpec((1, _D, 2 * _H),
                         lambda g, be_r: (be_r[jnp.minimum(g, be_r[_G] - 1)],
                                          0, 0)),
            pl.BlockSpec((1, _H, _D),
                         lambda g, be_r: (be_r[jnp.minimum(g, be_r[_G] - 1)],
                                          0, 0)),
        ],
        out_specs=pl.BlockSpec(
            (_BLK, _D), lambda g, be_r: (jnp.minimum(g, be_r[_G] - 1), 0)),
    )
    return pl.pallas_call(
        _gmm_body,
        grid_spec=grid_spec,
        out_shape=jax.ShapeDtypeStruct((_R, _D), jnp.float32),
    )(be, x_pad, gate_up_w, down_w)


def _shared_body(x_ref, gw_ref, uw_ref, dw_ref, moe_ref, wgt_ref, out_ref):
    s = pl.program_id(0)

    @pl.when(s == 0)
    def _init():
        out_ref[...] = moe_ref[...] * wgt_ref[...]

    g = jnp.dot(x_ref[...], gw_ref[...], preferred_element_type=jnp.float32)
    u = jnp.dot(x_ref[...], uw_ref[...], preferred_element_type=jnp.float32)
    a = jax.nn.silu(g) * u
    out_ref[...] += jnp.dot(a, dw_ref[...],
                            preferred_element_type=jnp.float32)


def _shared(xf, gw, uw, dw, moe_rows, wgt):
    ns = 2
    sb = _S // ns
    return pl.pallas_call(
        _shared_body,
        grid=(ns,),
        in_specs=[
            pl.BlockSpec((_T, _D), lambda s: (0, 0)),
            pl.BlockSpec((_D, sb), lambda s: (0, s)),
            pl.BlockSpec((_D, sb), lambda s: (0, s)),
            pl.BlockSpec((sb, _D), lambda s: (s, 0)),
            pl.BlockSpec((_T, _D), lambda s: (0, 0)),
            pl.BlockSpec((_T, 1), lambda s: (0, 0)),
        ],
        out_specs=pl.BlockSpec((_T, _D), lambda s: (0, 0)),
        out_shape=jax.ShapeDtypeStruct((_T, _D), jnp.float32),
    )(xf, gw, uw, dw, moe_rows, wgt)


@jax.jit
def kernel(x, router_w, gate_up_w, down_w, shared_gate_w, shared_up_w,
           shared_down_w):
    Bc, Tc, C = x.shape
    xf = x.reshape(Tc, C)

    dest, be, wgt = _plan(xf, router_w)
    dest_flat = dest.reshape(_T)
    be_flat = be.reshape(_G + 1)

    x_pad = _sc_scatter(xf, dest_flat)
    out_pad = _gmm(be_flat, x_pad, gate_up_w, down_w)
    moe_rows = _sc_gather(out_pad, dest_flat)

    out = _shared(xf, shared_gate_w, shared_up_w, shared_down_w,
                  moe_rows, wgt)
    return out.reshape(Bc, Tc, C)
